# baseline (speedup 1.0000x reference)
import functools

import jax
import jax.numpy as jnp
from jax import lax
from jax.experimental import pallas as pl
from jax.experimental.pallas import tpu as pltpu
from jax.experimental.pallas import tpu_sc as plsc

N = 10000
E = 320000
D = 128
H = 128
C = 64

NC = 2
NS = 16
NW = NC * NS

PAD_R = 8
NP = N + PAD_R
K = 80
NCHUNK = 128
PH = 64
EPT = NCHUNK * K
EP = NW * EPT

R_A = 632
R_LAST = N - 15 * R_A

_MESH = plsc.VectorSubcoreMesh(core_axis_name="c", subcore_axis_name="s")


def _deg_partials(src3, dst3, ones_k, zeros_nf):
    @functools.partial(
        pl.kernel,
        out_type=jax.ShapeDtypeStruct((NC, 2, N, H), jnp.float32),
        mesh=_MESH,
        scratch_types=[
            pltpu.VMEM((NCHUNK, K), jnp.int32),
            pltpu.VMEM((NCHUNK, K), jnp.int32),
            pltpu.VMEM((K, H), jnp.float32),
            pltpu.VMEM_SHARED((NP, H), jnp.float32),
        ],
    )
    def body(src_hbm, dst_hbm, ones_hbm, zeros_hbm, out_hbm,
             srcv, dstv, onesv, acc_sh):
        c = lax.axis_index("c")
        s = lax.axis_index("s")
        wid = c * NS + s
        pltpu.sync_copy(src_hbm.at[wid], srcv)
        pltpu.sync_copy(dst_hbm.at[wid], dstv)
        pltpu.sync_copy(ones_hbm, onesv)

        def zero_mine():
            @pl.when(s < 15)
            def _za():
                pltpu.sync_copy(zeros_hbm.at[pl.ds(s * R_A, R_A)],
                                acc_sh.at[pl.ds(s * R_A, R_A)])

            @pl.when(s == 15)
            def _zb():
                pltpu.sync_copy(zeros_hbm.at[pl.ds(15 * R_A, R_LAST)],
                                acc_sh.at[pl.ds(15 * R_A, R_LAST)])

        def copy_mine(which):
            @pl.when(s < 15)
            def _oa():
                pltpu.sync_copy(acc_sh.at[pl.ds(s * R_A, R_A)],
                                out_hbm.at[c, which].at[pl.ds(s * R_A, R_A)])

            @pl.when(s == 15)
            def _ob():
                pltpu.sync_copy(acc_sh.at[pl.ds(15 * R_A, R_LAST)],
                                out_hbm.at[c, which].at[pl.ds(15 * R_A, R_LAST)])

        def scatter_ones(idxv):
            def step(g, carry):
                pltpu.sync_copy(onesv, acc_sh.at[idxv.at[g]], add=True)
                return carry

            lax.fori_loop(0, NCHUNK, step, 0)

        zero_mine()
        plsc.subcore_barrier()
        scatter_ones(srcv)
        plsc.subcore_barrier()
        copy_mine(0)
        zero_mine()
        plsc.subcore_barrier()
        scatter_ones(dstv)
        plsc.subcore_barrier()
        copy_mine(1)

    return body(src3, dst3, ones_k, zeros_nf)


def _agg_partials(hw, src3, dst3, zeros_nf, F):
    @functools.partial(
        pl.kernel,
        out_type=jax.ShapeDtypeStruct((NC, N, F), jnp.float32),
        mesh=_MESH,
        scratch_types=[
            pltpu.VMEM((PH, K), jnp.int32),
            pltpu.VMEM((PH, K), jnp.int32),
            pltpu.VMEM((2, K, F), jnp.float32),
            pltpu.VMEM_SHARED((NP, F), jnp.float32),
            pltpu.SemaphoreType.DMA,
        ],
    )
    def body(hw_hbm, src_hbm, dst_hbm, zeros_hbm, out_hbm,
             srcv, dstv, rows, acc_sh, sem):
        c = lax.axis_index("c")
        s = lax.axis_index("s")
        wid = c * NS + s

        @pl.when(s < 15)
        def _zero_a():
            pltpu.sync_copy(zeros_hbm.at[pl.ds(s * R_A, R_A)],
                            acc_sh.at[pl.ds(s * R_A, R_A)])

        @pl.when(s == 15)
        def _zero_b():
            pltpu.sync_copy(zeros_hbm.at[pl.ds(15 * R_A, R_LAST)],
                            acc_sh.at[pl.ds(15 * R_A, R_LAST)])

        plsc.subcore_barrier()

        for p in range(NCHUNK // PH):
            pltpu.sync_copy(src_hbm.at[wid].at[pl.ds(p * PH, PH)], srcv)
            pltpu.sync_copy(dst_hbm.at[wid].at[pl.ds(p * PH, PH)], dstv)
            pltpu.async_copy(hw_hbm.at[srcv.at[0]], rows.at[0], sem)

            def step(g, carry):
                b = lax.rem(g, 2)
                pltpu.make_async_copy(hw_hbm.at[srcv.at[0]],
                                      rows.at[b], sem).wait()

                @pl.when(g + 1 < PH)
                def _next():
                    pltpu.async_copy(hw_hbm.at[srcv.at[g + 1]],
                                     rows.at[1 - b], sem)

                pltpu.sync_copy(rows.at[b], acc_sh.at[dstv.at[g]], add=True)
                return carry

            lax.fori_loop(0, PH, step, 0)
        plsc.subcore_barrier()

        @pl.when(s < 15)
        def _out_a():
            pltpu.sync_copy(acc_sh.at[pl.ds(s * R_A, R_A)],
                            out_hbm.at[c].at[pl.ds(s * R_A, R_A)])

        @pl.when(s == 15)
        def _out_b():
            pltpu.sync_copy(acc_sh.at[pl.ds(15 * R_A, R_LAST)],
                            out_hbm.at[c].at[pl.ds(15 * R_A, R_LAST)])

    return body(hw, src3, dst3, zeros_nf)


def _dense_first(degT, featp, W1):
    def body(degT_ref, feat_ref, w1_ref, ns_ref, nd_ref, hw_ref):
        deg_out = degT_ref[0, :, 0:1] + degT_ref[2, :, 0:1]
        deg_in = degT_ref[1, :, 0:1] + degT_ref[3, :, 0:1]
        ns = lax.rsqrt(jnp.maximum(deg_out, 1.0))
        nd = lax.rsqrt(jnp.maximum(deg_in, 1.0))
        ns_b = jnp.broadcast_to(ns, (N, H))
        nd_b = jnp.broadcast_to(nd, (N, H))
        ns_ref[...] = ns_b
        nd_ref[...] = nd_b
        feat = feat_ref[...][:N]
        hw_ref[...] = jnp.concatenate(
            [jnp.dot(feat * ns_b, w1_ref[...],
                     preferred_element_type=jnp.float32),
             jnp.zeros((PAD_R, H), jnp.float32)], axis=0)

    return pl.pallas_call(
        body,
        out_shape=(
            jax.ShapeDtypeStruct((N, H), jnp.float32),
            jax.ShapeDtypeStruct((N, H), jnp.float32),
            jax.ShapeDtypeStruct((NP, H), jnp.float32),
        ),
    )(degT, featp, W1)


def _dense_mid(aggp, ns_b, nd_b, W):
    Fo = W.shape[1]

    def body(aggp_ref, ns_ref, nd_ref, w_ref, out_ref):
        agg = aggp_ref[0] + aggp_ref[1]
        h = jax.nn.relu(agg * nd_ref[...]) * ns_ref[...]
        out_ref[...] = jnp.concatenate(
            [jnp.dot(h, w_ref[...], preferred_element_type=jnp.float32),
             jnp.zeros((PAD_R, Fo), jnp.float32)], axis=0)

    return pl.pallas_call(
        body,
        out_shape=jax.ShapeDtypeStruct((NP, Fo), jnp.float32),
    )(aggp, ns_b, nd_b, W)


def _dense_act(aggp, ns_b, nd_b):
    def body(aggp_ref, ns_ref, nd_ref, out_ref):
        agg = aggp_ref[0] + aggp_ref[1]
        h = jax.nn.relu(agg * nd_ref[...]) * ns_ref[...]
        out_ref[...] = jnp.concatenate(
            [h, jnp.zeros((PAD_R, H), jnp.float32)], axis=0)

    return pl.pallas_call(
        body,
        out_shape=jax.ShapeDtypeStruct((NP, H), jnp.float32),
    )(aggp, ns_b, nd_b)


def _dense_out(aggp, nd_b, W3, b3):
    def body(aggp_ref, nd_ref, w_ref, b_ref, out_ref):
        agg = aggp_ref[0] + aggp_ref[1]
        mm = jnp.dot(agg, w_ref[...], preferred_element_type=jnp.float32)
        out_ref[...] = mm * nd_ref[:, :C] + b_ref[...][None, :]

    return pl.pallas_call(
        body,
        out_shape=jax.ShapeDtypeStruct((N, C), jnp.float32),
    )(aggp, nd_b, W3, b3)


def kernel(feat, edge_index, W1, W2, W3, b3):
    pad = jnp.full((EP - E,), N, jnp.int32)
    src3 = jnp.concatenate([edge_index[0], pad]).reshape(NW, NCHUNK, K)
    dst3 = jnp.concatenate([edge_index[1], pad]).reshape(NW, NCHUNK, K)
    featp = jnp.concatenate([feat, jnp.zeros((PAD_R, D), jnp.float32)])

    ones_k = jnp.ones((K, H), jnp.float32)
    z128 = jnp.zeros((N, H), jnp.float32)

    degp = _deg_partials(src3, dst3, ones_k, z128)
    degT = degp.reshape(2 * 2, N, H)
    ns_b, nd_b, hw1 = _dense_first(degT, featp, W1)
    aggp1 = _agg_partials(hw1, src3, dst3, z128, H)
    hw2 = _dense_mid(aggp1, ns_b, nd_b, W2)
    aggp2 = _agg_partials(hw2, src3, dst3, z128, H)
    u3 = _dense_act(aggp2, ns_b, nd_b)
    aggp3 = _agg_partials(u3, src3, dst3, z128, H)
    return _dense_out(aggp3, nd_b, W3, b3)


# speedup vs baseline: 2.5724x; 2.5724x over previous
import functools

import jax
import jax.numpy as jnp
from jax import lax
from jax.experimental import pallas as pl
from jax.experimental.pallas import tpu as pltpu
from jax.experimental.pallas import tpu_sc as plsc

N = 10000
E = 320000
D = 128
H = 128
C = 64

NC = 2
NS = 16
NW = NC * NS

PAD_R = 128
NP = N + PAD_R
K = 80
NCHUNK = 128
PH = 64
EPT = NCHUNK * K
EP = NW * EPT

R_A = 632
R_LAST = N - 15 * R_A

_MESH = plsc.VectorSubcoreMesh(core_axis_name="c", subcore_axis_name="s")


def _deg_partials(src3, dst3, ones_k, zeros_nf):
    @functools.partial(
        pl.kernel,
        out_type=jax.ShapeDtypeStruct((NC, 2, N, H), jnp.float32),
        mesh=_MESH,
        scratch_types=[
            pltpu.VMEM((NCHUNK, K), jnp.int32),
            pltpu.VMEM((NCHUNK, K), jnp.int32),
            pltpu.VMEM((K, H), jnp.float32),
            pltpu.VMEM_SHARED((NP, H), jnp.float32),
        ],
    )
    def body(src_hbm, dst_hbm, ones_hbm, zeros_hbm, out_hbm,
             srcv, dstv, onesv, acc_sh):
        c = lax.axis_index("c")
        s = lax.axis_index("s")
        wid = c * NS + s
        pltpu.sync_copy(src_hbm.at[wid], srcv)
        pltpu.sync_copy(dst_hbm.at[wid], dstv)
        pltpu.sync_copy(ones_hbm, onesv)

        def zero_mine():
            @pl.when(s < 15)
            def _za():
                pltpu.sync_copy(zeros_hbm.at[pl.ds(s * R_A, R_A)],
                                acc_sh.at[pl.ds(s * R_A, R_A)])

            @pl.when(s == 15)
            def _zb():
                pltpu.sync_copy(zeros_hbm.at[pl.ds(15 * R_A, R_LAST)],
                                acc_sh.at[pl.ds(15 * R_A, R_LAST)])

        def copy_mine(which):
            @pl.when(s < 15)
            def _oa():
                pltpu.sync_copy(acc_sh.at[pl.ds(s * R_A, R_A)],
                                out_hbm.at[c, which].at[pl.ds(s * R_A, R_A)])

            @pl.when(s == 15)
            def _ob():
                pltpu.sync_copy(acc_sh.at[pl.ds(15 * R_A, R_LAST)],
                                out_hbm.at[c, which].at[pl.ds(15 * R_A, R_LAST)])

        def scatter_ones(idxv):
            def step(g, carry):
                pltpu.sync_copy(onesv, acc_sh.at[idxv.at[g]], add=True)
                return carry

            lax.fori_loop(0, NCHUNK, step, 0)

        zero_mine()
        plsc.subcore_barrier()
        scatter_ones(srcv)
        plsc.subcore_barrier()
        copy_mine(0)
        zero_mine()
        plsc.subcore_barrier()
        scatter_ones(dstv)
        plsc.subcore_barrier()
        copy_mine(1)

    return body(src3, dst3, ones_k, zeros_nf)


def _agg_partials(hw, src3, dst3, zeros_nf, F):
    @functools.partial(
        pl.kernel,
        out_type=jax.ShapeDtypeStruct((NC, N, F), jnp.float32),
        mesh=_MESH,
        scratch_types=[
            pltpu.VMEM((PH, K), jnp.int32),
            pltpu.VMEM((PH, K), jnp.int32),
            pltpu.VMEM((2, K, F), jnp.float32),
            pltpu.VMEM_SHARED((NP, F), jnp.float32),
            pltpu.SemaphoreType.DMA,
        ],
    )
    def body(hw_hbm, src_hbm, dst_hbm, zeros_hbm, out_hbm,
             srcv, dstv, rows, acc_sh, sem):
        c = lax.axis_index("c")
        s = lax.axis_index("s")
        wid = c * NS + s

        @pl.when(s < 15)
        def _zero_a():
            pltpu.sync_copy(zeros_hbm.at[pl.ds(s * R_A, R_A)],
                            acc_sh.at[pl.ds(s * R_A, R_A)])

        @pl.when(s == 15)
        def _zero_b():
            pltpu.sync_copy(zeros_hbm.at[pl.ds(15 * R_A, R_LAST)],
                            acc_sh.at[pl.ds(15 * R_A, R_LAST)])

        plsc.subcore_barrier()

        for p in range(NCHUNK // PH):
            pltpu.sync_copy(src_hbm.at[wid].at[pl.ds(p * PH, PH)], srcv)
            pltpu.sync_copy(dst_hbm.at[wid].at[pl.ds(p * PH, PH)], dstv)
            pltpu.async_copy(hw_hbm.at[srcv.at[0]], rows.at[0], sem)

            def step(g, carry):
                b = lax.rem(g, 2)
                pltpu.make_async_copy(hw_hbm.at[srcv.at[0]],
                                      rows.at[b], sem).wait()

                @pl.when(g + 1 < PH)
                def _next():
                    pltpu.async_copy(hw_hbm.at[srcv.at[g + 1]],
                                     rows.at[1 - b], sem)

                pltpu.sync_copy(rows.at[b], acc_sh.at[dstv.at[g]], add=True)
                return carry

            lax.fori_loop(0, PH, step, 0)
        plsc.subcore_barrier()

        @pl.when(s < 15)
        def _out_a():
            pltpu.sync_copy(acc_sh.at[pl.ds(s * R_A, R_A)],
                            out_hbm.at[c].at[pl.ds(s * R_A, R_A)])

        @pl.when(s == 15)
        def _out_b():
            pltpu.sync_copy(acc_sh.at[pl.ds(15 * R_A, R_LAST)],
                            out_hbm.at[c].at[pl.ds(15 * R_A, R_LAST)])

    return body(hw, src3, dst3, zeros_nf)


def _dense_first(degT, featp, W1):
    def body(degT_ref, feat_ref, w1_ref, ns_ref, nd_ref, hw_ref):
        deg_out = degT_ref[0, :, 0:1] + degT_ref[2, :, 0:1]
        deg_in = degT_ref[1, :, 0:1] + degT_ref[3, :, 0:1]
        ns = lax.rsqrt(jnp.maximum(deg_out, 1.0))
        nd = lax.rsqrt(jnp.maximum(deg_in, 1.0))
        ns_b = jnp.broadcast_to(ns, (N, H))
        nd_b = jnp.broadcast_to(nd, (N, H))
        ns_ref[...] = ns_b
        nd_ref[...] = nd_b
        feat = feat_ref[...][:N]
        hw_ref[...] = jnp.concatenate(
            [jnp.dot(feat * ns_b, w1_ref[...],
                     preferred_element_type=jnp.float32),
             jnp.zeros((PAD_R, H), jnp.float32)], axis=0)

    return pl.pallas_call(
        body,
        out_shape=(
            jax.ShapeDtypeStruct((N, H), jnp.float32),
            jax.ShapeDtypeStruct((N, H), jnp.float32),
            jax.ShapeDtypeStruct((NP, H), jnp.float32),
        ),
    )(degT, featp, W1)


def _dense_mid(aggp, ns_b, nd_b, W):
    Fo = W.shape[1]

    def body(aggp_ref, ns_ref, nd_ref, w_ref, out_ref):
        agg = aggp_ref[0] + aggp_ref[1]
        h = jax.nn.relu(agg * nd_ref[...]) * ns_ref[...]
        out_ref[...] = jnp.concatenate(
            [jnp.dot(h, w_ref[...], preferred_element_type=jnp.float32),
             jnp.zeros((PAD_R, Fo), jnp.float32)], axis=0)

    return pl.pallas_call(
        body,
        out_shape=jax.ShapeDtypeStruct((NP, Fo), jnp.float32),
    )(aggp, ns_b, nd_b, W)


def _dense_act(aggp, ns_b, nd_b):
    def body(aggp_ref, ns_ref, nd_ref, out_ref):
        agg = aggp_ref[0] + aggp_ref[1]
        h = jax.nn.relu(agg * nd_ref[...]) * ns_ref[...]
        out_ref[...] = jnp.concatenate(
            [h, jnp.zeros((PAD_R, H), jnp.float32)], axis=0)

    return pl.pallas_call(
        body,
        out_shape=jax.ShapeDtypeStruct((NP, H), jnp.float32),
    )(aggp, ns_b, nd_b)


def _dense_out(aggp, nd_b, W3, b3):
    def body(aggp_ref, nd_ref, w_ref, b_ref, out_ref):
        agg = aggp_ref[0] + aggp_ref[1]
        mm = jnp.dot(agg, w_ref[...], preferred_element_type=jnp.float32)
        out_ref[...] = mm * nd_ref[:, :C] + b_ref[...][None, :]

    return pl.pallas_call(
        body,
        out_shape=jax.ShapeDtypeStruct((N, C), jnp.float32),
    )(aggp, nd_b, W3, b3)


def kernel(feat, edge_index, W1, W2, W3, b3):
    pad = N + (jnp.arange(EP - E, dtype=jnp.int32) % PAD_R)
    src3 = jnp.concatenate([edge_index[0], pad]).reshape(NW, NCHUNK, K)
    dst3 = jnp.concatenate([edge_index[1], pad]).reshape(NW, NCHUNK, K)
    featp = jnp.concatenate([feat, jnp.zeros((PAD_R, D), jnp.float32)])

    ones_k = jnp.ones((K, H), jnp.float32)
    z128 = jnp.zeros((N, H), jnp.float32)

    degp = _deg_partials(src3, dst3, ones_k, z128)
    degT = degp.reshape(2 * 2, N, H)
    ns_b, nd_b, hw1 = _dense_first(degT, featp, W1)
    aggp1 = _agg_partials(hw1, src3, dst3, z128, H)
    hw2 = _dense_mid(aggp1, ns_b, nd_b, W2)
    aggp2 = _agg_partials(hw2, src3, dst3, z128, H)
    u3 = _dense_act(aggp2, ns_b, nd_b)
    aggp3 = _agg_partials(u3, src3, dst3, z128, H)
    return _dense_out(aggp3, nd_b, W3, b3)


# speedup vs baseline: 2.9455x; 1.1450x over previous
import functools

import jax
import jax.numpy as jnp
from jax import lax
from jax.experimental import pallas as pl
from jax.experimental.pallas import tpu as pltpu
from jax.experimental.pallas import tpu_sc as plsc

N = 10000
E = 320000
D = 128
H = 128
C = 64

NC = 2
NS = 16
NW = NC * NS

PAD_R = 128
NP = N + PAD_R
K = 128
NCHUNK = 80
PH = 40
EPT = NCHUNK * K
EP = NW * EPT

R_A = 632
R_LAST = N - 15 * R_A

_MESH = plsc.VectorSubcoreMesh(core_axis_name="c", subcore_axis_name="s")


def _deg_partials(src3, dst3, ones_k, zeros_nf):
    @functools.partial(
        pl.kernel,
        out_type=jax.ShapeDtypeStruct((NC, 2, N, H), jnp.float32),
        mesh=_MESH,
        scratch_types=[
            pltpu.VMEM((NCHUNK, K), jnp.int32),
            pltpu.VMEM((NCHUNK, K), jnp.int32),
            pltpu.VMEM((K, H), jnp.float32),
            pltpu.VMEM_SHARED((NP, H), jnp.float32),
        ],
    )
    def body(src_hbm, dst_hbm, ones_hbm, zeros_hbm, out_hbm,
             srcv, dstv, onesv, acc_sh):
        c = lax.axis_index("c")
        s = lax.axis_index("s")
        wid = c * NS + s
        pltpu.sync_copy(src_hbm.at[wid], srcv)
        pltpu.sync_copy(dst_hbm.at[wid], dstv)
        pltpu.sync_copy(ones_hbm, onesv)

        def zero_mine():
            @pl.when(s < 15)
            def _za():
                pltpu.sync_copy(zeros_hbm.at[pl.ds(s * R_A, R_A)],
                                acc_sh.at[pl.ds(s * R_A, R_A)])

            @pl.when(s == 15)
            def _zb():
                pltpu.sync_copy(zeros_hbm.at[pl.ds(15 * R_A, R_LAST)],
                                acc_sh.at[pl.ds(15 * R_A, R_LAST)])

        def copy_mine(which):
            @pl.when(s < 15)
            def _oa():
                pltpu.sync_copy(acc_sh.at[pl.ds(s * R_A, R_A)],
                                out_hbm.at[c, which].at[pl.ds(s * R_A, R_A)])

            @pl.when(s == 15)
            def _ob():
                pltpu.sync_copy(acc_sh.at[pl.ds(15 * R_A, R_LAST)],
                                out_hbm.at[c, which].at[pl.ds(15 * R_A, R_LAST)])

        def scatter_ones(idxv):
            def step(g, carry):
                pltpu.sync_copy(onesv, acc_sh.at[idxv.at[g]], add=True)
                return carry

            lax.fori_loop(0, NCHUNK, step, 0)

        zero_mine()
        plsc.subcore_barrier()
        scatter_ones(srcv)
        plsc.subcore_barrier()
        copy_mine(0)
        zero_mine()
        plsc.subcore_barrier()
        scatter_ones(dstv)
        plsc.subcore_barrier()
        copy_mine(1)

    return body(src3, dst3, ones_k, zeros_nf)


def _agg_partials(hw, src3, dst3, zeros_nf, F):
    @functools.partial(
        pl.kernel,
        out_type=jax.ShapeDtypeStruct((NC, N, F), jnp.float32),
        mesh=_MESH,
        scratch_types=[
            pltpu.VMEM((PH, K), jnp.int32),
            pltpu.VMEM((PH, K), jnp.int32),
            pltpu.VMEM((2, K, F), jnp.float32),
            pltpu.VMEM_SHARED((NP, F), jnp.float32),
            pltpu.SemaphoreType.DMA,
        ],
    )
    def body(hw_hbm, src_hbm, dst_hbm, zeros_hbm, out_hbm,
             srcv, dstv, rows, acc_sh, sem):
        c = lax.axis_index("c")
        s = lax.axis_index("s")
        wid = c * NS + s

        @pl.when(s < 15)
        def _zero_a():
            pltpu.sync_copy(zeros_hbm.at[pl.ds(s * R_A, R_A)],
                            acc_sh.at[pl.ds(s * R_A, R_A)])

        @pl.when(s == 15)
        def _zero_b():
            pltpu.sync_copy(zeros_hbm.at[pl.ds(15 * R_A, R_LAST)],
                            acc_sh.at[pl.ds(15 * R_A, R_LAST)])

        plsc.subcore_barrier()

        for p in range(NCHUNK // PH):
            pltpu.sync_copy(src_hbm.at[wid].at[pl.ds(p * PH, PH)], srcv)
            pltpu.sync_copy(dst_hbm.at[wid].at[pl.ds(p * PH, PH)], dstv)
            pltpu.async_copy(hw_hbm.at[srcv.at[0]], rows.at[0], sem)

            def step(g, carry):
                b = lax.rem(g, 2)
                pltpu.make_async_copy(hw_hbm.at[srcv.at[0]],
                                      rows.at[b], sem).wait()

                @pl.when(g + 1 < PH)
                def _next():
                    pltpu.async_copy(hw_hbm.at[srcv.at[g + 1]],
                                     rows.at[1 - b], sem)

                pltpu.sync_copy(rows.at[b], acc_sh.at[dstv.at[g]], add=True)
                return carry

            lax.fori_loop(0, PH, step, 0)
        plsc.subcore_barrier()

        @pl.when(s < 15)
        def _out_a():
            pltpu.sync_copy(acc_sh.at[pl.ds(s * R_A, R_A)],
                            out_hbm.at[c].at[pl.ds(s * R_A, R_A)])

        @pl.when(s == 15)
        def _out_b():
            pltpu.sync_copy(acc_sh.at[pl.ds(15 * R_A, R_LAST)],
                            out_hbm.at[c].at[pl.ds(15 * R_A, R_LAST)])

    return body(hw, src3, dst3, zeros_nf)


def _dense_first(degT, featp, W1):
    def body(degT_ref, feat_ref, w1_ref, ns_ref, nd_ref, hw_ref):
        deg_out = degT_ref[0, :, 0:1] + degT_ref[2, :, 0:1]
        deg_in = degT_ref[1, :, 0:1] + degT_ref[3, :, 0:1]
        ns = lax.rsqrt(jnp.maximum(deg_out, 1.0))
        nd = lax.rsqrt(jnp.maximum(deg_in, 1.0))
        ns_b = jnp.broadcast_to(ns, (N, H))
        nd_b = jnp.broadcast_to(nd, (N, H))
        ns_ref[...] = ns_b
        nd_ref[...] = nd_b
        feat = feat_ref[...][:N]
        hw_ref[...] = jnp.concatenate(
            [jnp.dot(feat * ns_b, w1_ref[...],
                     preferred_element_type=jnp.float32),
             jnp.zeros((PAD_R, H), jnp.float32)], axis=0)

    return pl.pallas_call(
        body,
        out_shape=(
            jax.ShapeDtypeStruct((N, H), jnp.float32),
            jax.ShapeDtypeStruct((N, H), jnp.float32),
            jax.ShapeDtypeStruct((NP, H), jnp.float32),
        ),
    )(degT, featp, W1)


def _dense_mid(aggp, ns_b, nd_b, W):
    Fo = W.shape[1]

    def body(aggp_ref, ns_ref, nd_ref, w_ref, out_ref):
        agg = aggp_ref[0] + aggp_ref[1]
        h = jax.nn.relu(agg * nd_ref[...]) * ns_ref[...]
        out_ref[...] = jnp.concatenate(
            [jnp.dot(h, w_ref[...], preferred_element_type=jnp.float32),
             jnp.zeros((PAD_R, Fo), jnp.float32)], axis=0)

    return pl.pallas_call(
        body,
        out_shape=jax.ShapeDtypeStruct((NP, Fo), jnp.float32),
    )(aggp, ns_b, nd_b, W)


def _dense_act(aggp, ns_b, nd_b):
    def body(aggp_ref, ns_ref, nd_ref, out_ref):
        agg = aggp_ref[0] + aggp_ref[1]
        h = jax.nn.relu(agg * nd_ref[...]) * ns_ref[...]
        out_ref[...] = jnp.concatenate(
            [h, jnp.zeros((PAD_R, H), jnp.float32)], axis=0)

    return pl.pallas_call(
        body,
        out_shape=jax.ShapeDtypeStruct((NP, H), jnp.float32),
    )(aggp, ns_b, nd_b)


def _dense_out(aggp, nd_b, W3, b3):
    def body(aggp_ref, nd_ref, w_ref, b_ref, out_ref):
        agg = aggp_ref[0] + aggp_ref[1]
        mm = jnp.dot(agg, w_ref[...], preferred_element_type=jnp.float32)
        out_ref[...] = mm * nd_ref[:, :C] + b_ref[...][None, :]

    return pl.pallas_call(
        body,
        out_shape=jax.ShapeDtypeStruct((N, C), jnp.float32),
    )(aggp, nd_b, W3, b3)


def kernel(feat, edge_index, W1, W2, W3, b3):
    pad = N + (jnp.arange(EP - E, dtype=jnp.int32) % PAD_R)
    src3 = jnp.concatenate([edge_index[0], pad]).reshape(NW, NCHUNK, K)
    dst3 = jnp.concatenate([edge_index[1], pad]).reshape(NW, NCHUNK, K)
    featp = jnp.concatenate([feat, jnp.zeros((PAD_R, D), jnp.float32)])

    ones_k = jnp.ones((K, H), jnp.float32)
    z128 = jnp.zeros((N, H), jnp.float32)

    degp = _deg_partials(src3, dst3, ones_k, z128)
    degT = degp.reshape(2 * 2, N, H)
    ns_b, nd_b, hw1 = _dense_first(degT, featp, W1)
    aggp1 = _agg_partials(hw1, src3, dst3, z128, H)
    hw2 = _dense_mid(aggp1, ns_b, nd_b, W2)
    aggp2 = _agg_partials(hw2, src3, dst3, z128, H)
    u3 = _dense_act(aggp2, ns_b, nd_b)
    aggp3 = _agg_partials(u3, src3, dst3, z128, H)
    return _dense_out(aggp3, nd_b, W3, b3)


# speedup vs baseline: 2.9463x; 1.0003x over previous
import functools

import jax
import jax.numpy as jnp
from jax import lax
from jax.experimental import pallas as pl
from jax.experimental.pallas import tpu as pltpu
from jax.experimental.pallas import tpu_sc as plsc

N = 10000
E = 320000
D = 128
H = 128
C = 64

NC = 2
NS = 16
NW = NC * NS

PAD_R = 128
NP = N + PAD_R
K = 128
NCHUNK = 80
PH = 40
EPT = NCHUNK * K
EP = NW * EPT

R_A = 632
R_LAST = N - 15 * R_A

_MESH = plsc.VectorSubcoreMesh(core_axis_name="c", subcore_axis_name="s")


def _deg_partials(src3, dst3, ones_k, zeros_nf):
    @functools.partial(
        pl.kernel,
        out_type=jax.ShapeDtypeStruct((NC, 2, N, H), jnp.float32),
        mesh=_MESH,
        scratch_types=[
            pltpu.VMEM((NCHUNK, K), jnp.int32),
            pltpu.VMEM((NCHUNK, K), jnp.int32),
            pltpu.VMEM((K, H), jnp.float32),
            pltpu.VMEM_SHARED((NP, H), jnp.float32),
            pltpu.SemaphoreType.DMA,
        ],
    )
    def body(src_hbm, dst_hbm, ones_hbm, zeros_hbm, out_hbm,
             srcv, dstv, onesv, acc_sh, sem):
        c = lax.axis_index("c")
        s = lax.axis_index("s")
        wid = c * NS + s
        pltpu.sync_copy(src_hbm.at[wid], srcv)
        pltpu.sync_copy(dst_hbm.at[wid], dstv)
        pltpu.sync_copy(ones_hbm, onesv)

        def zero_mine():
            @pl.when(s < 15)
            def _za():
                pltpu.sync_copy(zeros_hbm.at[pl.ds(s * R_A, R_A)],
                                acc_sh.at[pl.ds(s * R_A, R_A)])

            @pl.when(s == 15)
            def _zb():
                pltpu.sync_copy(zeros_hbm.at[pl.ds(15 * R_A, R_LAST)],
                                acc_sh.at[pl.ds(15 * R_A, R_LAST)])

        def copy_mine(which):
            @pl.when(s < 15)
            def _oa():
                pltpu.sync_copy(acc_sh.at[pl.ds(s * R_A, R_A)],
                                out_hbm.at[c, which].at[pl.ds(s * R_A, R_A)])

            @pl.when(s == 15)
            def _ob():
                pltpu.sync_copy(acc_sh.at[pl.ds(15 * R_A, R_LAST)],
                                out_hbm.at[c, which].at[pl.ds(15 * R_A, R_LAST)])

        def scatter_ones(idxv):
            def fire(g, carry):
                pltpu.async_copy(onesv, acc_sh.at[idxv.at[g]], sem, add=True)
                return carry

            lax.fori_loop(0, NCHUNK, fire, 0)

            def drain(g, carry):
                pltpu.make_async_copy(onesv, acc_sh.at[idxv.at[0]],
                                      sem).wait()
                return carry

            lax.fori_loop(0, NCHUNK, drain, 0)

        zero_mine()
        plsc.subcore_barrier()
        scatter_ones(srcv)
        plsc.subcore_barrier()
        copy_mine(0)
        zero_mine()
        plsc.subcore_barrier()
        scatter_ones(dstv)
        plsc.subcore_barrier()
        copy_mine(1)

    return body(src3, dst3, ones_k, zeros_nf)


def _agg_partials(hw, src3, dst3, zeros_nf, F):
    @functools.partial(
        pl.kernel,
        out_type=jax.ShapeDtypeStruct((NC, N, F), jnp.float32),
        mesh=_MESH,
        scratch_types=[
            pltpu.VMEM((PH, K), jnp.int32),
            pltpu.VMEM((PH, K), jnp.int32),
            pltpu.VMEM((2, K, F), jnp.float32),
            pltpu.VMEM_SHARED((NP, F), jnp.float32),
            pltpu.SemaphoreType.DMA,
        ],
    )
    def body(hw_hbm, src_hbm, dst_hbm, zeros_hbm, out_hbm,
             srcv, dstv, rows, acc_sh, sem):
        c = lax.axis_index("c")
        s = lax.axis_index("s")
        wid = c * NS + s

        @pl.when(s < 15)
        def _zero_a():
            pltpu.sync_copy(zeros_hbm.at[pl.ds(s * R_A, R_A)],
                            acc_sh.at[pl.ds(s * R_A, R_A)])

        @pl.when(s == 15)
        def _zero_b():
            pltpu.sync_copy(zeros_hbm.at[pl.ds(15 * R_A, R_LAST)],
                            acc_sh.at[pl.ds(15 * R_A, R_LAST)])

        plsc.subcore_barrier()

        for p in range(NCHUNK // PH):
            pltpu.sync_copy(src_hbm.at[wid].at[pl.ds(p * PH, PH)], srcv)
            pltpu.sync_copy(dst_hbm.at[wid].at[pl.ds(p * PH, PH)], dstv)
            pltpu.async_copy(hw_hbm.at[srcv.at[0]], rows.at[0], sem)

            def step(g, carry):
                b = lax.rem(g, 2)
                pltpu.make_async_copy(hw_hbm.at[srcv.at[0]],
                                      rows.at[b], sem).wait()

                @pl.when(g + 1 < PH)
                def _next():
                    pltpu.async_copy(hw_hbm.at[srcv.at[g + 1]],
                                     rows.at[1 - b], sem)

                pltpu.sync_copy(rows.at[b], acc_sh.at[dstv.at[g]], add=True)
                return carry

            lax.fori_loop(0, PH, step, 0)
        plsc.subcore_barrier()

        @pl.when(s < 15)
        def _out_a():
            pltpu.sync_copy(acc_sh.at[pl.ds(s * R_A, R_A)],
                            out_hbm.at[c].at[pl.ds(s * R_A, R_A)])

        @pl.when(s == 15)
        def _out_b():
            pltpu.sync_copy(acc_sh.at[pl.ds(15 * R_A, R_LAST)],
                            out_hbm.at[c].at[pl.ds(15 * R_A, R_LAST)])

    return body(hw, src3, dst3, zeros_nf)


def _dense_first(degT, featp, W1):
    def body(degT_ref, feat_ref, w1_ref, ns_ref, nd_ref, hw_ref):
        deg_out = degT_ref[0, :, 0:1] + degT_ref[2, :, 0:1]
        deg_in = degT_ref[1, :, 0:1] + degT_ref[3, :, 0:1]
        ns = lax.rsqrt(jnp.maximum(deg_out, 1.0))
        nd = lax.rsqrt(jnp.maximum(deg_in, 1.0))
        ns_b = jnp.broadcast_to(ns, (N, H))
        nd_b = jnp.broadcast_to(nd, (N, H))
        ns_ref[...] = ns_b
        nd_ref[...] = nd_b
        feat = feat_ref[...][:N]
        hw_ref[...] = jnp.concatenate(
            [jnp.dot(feat * ns_b, w1_ref[...],
                     preferred_element_type=jnp.float32),
             jnp.zeros((PAD_R, H), jnp.float32)], axis=0)

    return pl.pallas_call(
        body,
        out_shape=(
            jax.ShapeDtypeStruct((N, H), jnp.float32),
            jax.ShapeDtypeStruct((N, H), jnp.float32),
            jax.ShapeDtypeStruct((NP, H), jnp.float32),
        ),
    )(degT, featp, W1)


def _dense_mid(aggp, ns_b, nd_b, W):
    Fo = W.shape[1]

    def body(aggp_ref, ns_ref, nd_ref, w_ref, out_ref):
        agg = aggp_ref[0] + aggp_ref[1]
        h = jax.nn.relu(agg * nd_ref[...]) * ns_ref[...]
        out_ref[...] = jnp.concatenate(
            [jnp.dot(h, w_ref[...], preferred_element_type=jnp.float32),
             jnp.zeros((PAD_R, Fo), jnp.float32)], axis=0)

    return pl.pallas_call(
        body,
        out_shape=jax.ShapeDtypeStruct((NP, Fo), jnp.float32),
    )(aggp, ns_b, nd_b, W)


def _dense_act(aggp, ns_b, nd_b):
    def body(aggp_ref, ns_ref, nd_ref, out_ref):
        agg = aggp_ref[0] + aggp_ref[1]
        h = jax.nn.relu(agg * nd_ref[...]) * ns_ref[...]
        out_ref[...] = jnp.concatenate(
            [h, jnp.zeros((PAD_R, H), jnp.float32)], axis=0)

    return pl.pallas_call(
        body,
        out_shape=jax.ShapeDtypeStruct((NP, H), jnp.float32),
    )(aggp, ns_b, nd_b)


def _dense_out(aggp, nd_b, W3, b3):
    def body(aggp_ref, nd_ref, w_ref, b_ref, out_ref):
        agg = aggp_ref[0] + aggp_ref[1]
        mm = jnp.dot(agg, w_ref[...], preferred_element_type=jnp.float32)
        out_ref[...] = mm * nd_ref[:, :C] + b_ref[...][None, :]

    return pl.pallas_call(
        body,
        out_shape=jax.ShapeDtypeStruct((N, C), jnp.float32),
    )(aggp, nd_b, W3, b3)


def kernel(feat, edge_index, W1, W2, W3, b3):
    pad = N + (jnp.arange(EP - E, dtype=jnp.int32) % PAD_R)
    src3 = jnp.concatenate([edge_index[0], pad]).reshape(NW, NCHUNK, K)
    dst3 = jnp.concatenate([edge_index[1], pad]).reshape(NW, NCHUNK, K)
    featp = jnp.concatenate([feat, jnp.zeros((PAD_R, D), jnp.float32)])

    ones_k = jnp.ones((K, H), jnp.float32)
    z128 = jnp.zeros((N, H), jnp.float32)

    degp = _deg_partials(src3, dst3, ones_k, z128)
    degT = degp.reshape(2 * 2, N, H)
    ns_b, nd_b, hw1 = _dense_first(degT, featp, W1)
    aggp1 = _agg_partials(hw1, src3, dst3, z128, H)
    hw2 = _dense_mid(aggp1, ns_b, nd_b, W2)
    aggp2 = _agg_partials(hw2, src3, dst3, z128, H)
    u3 = _dense_act(aggp2, ns_b, nd_b)
    aggp3 = _agg_partials(u3, src3, dst3, z128, H)
    return _dense_out(aggp3, nd_b, W3, b3)


# speedup vs baseline: 2.9529x; 1.0022x over previous
import functools

import jax
import jax.numpy as jnp
from jax import lax
from jax.experimental import pallas as pl
from jax.experimental.pallas import tpu as pltpu
from jax.experimental.pallas import tpu_sc as plsc

N = 10000
E = 320000
D = 128
H = 128
C = 64

NC = 2
NS = 16
NW = NC * NS

PAD_R = 128
NP = N + PAD_R
K = 128
NCHUNK = 80
PH = 40
EPT = NCHUNK * K
EP = NW * EPT

R_A = 632
R_LAST = N - 15 * R_A

_MESH = plsc.VectorSubcoreMesh(core_axis_name="c", subcore_axis_name="s")


def _deg_partials(src3, dst3, ones_k, zeros_nf):
    @functools.partial(
        pl.kernel,
        out_type=jax.ShapeDtypeStruct((NC, 2, N, H), jnp.float32),
        mesh=_MESH,
        scratch_types=[
            pltpu.VMEM((NCHUNK, K), jnp.int32),
            pltpu.VMEM((NCHUNK, K), jnp.int32),
            pltpu.VMEM((K, H), jnp.float32),
            pltpu.VMEM_SHARED((NP, H), jnp.float32),
            pltpu.SemaphoreType.DMA,
        ],
    )
    def body(src_hbm, dst_hbm, ones_hbm, zeros_hbm, out_hbm,
             srcv, dstv, onesv, acc_sh, sem):
        c = lax.axis_index("c")
        s = lax.axis_index("s")
        wid = c * NS + s
        pltpu.sync_copy(src_hbm.at[wid], srcv)
        pltpu.sync_copy(dst_hbm.at[wid], dstv)
        pltpu.sync_copy(ones_hbm, onesv)

        def zero_mine():
            @pl.when(s < 15)
            def _za():
                pltpu.sync_copy(zeros_hbm.at[pl.ds(s * R_A, R_A)],
                                acc_sh.at[pl.ds(s * R_A, R_A)])

            @pl.when(s == 15)
            def _zb():
                pltpu.sync_copy(zeros_hbm.at[pl.ds(15 * R_A, R_LAST)],
                                acc_sh.at[pl.ds(15 * R_A, R_LAST)])

        def copy_mine(which):
            @pl.when(s < 15)
            def _oa():
                pltpu.sync_copy(acc_sh.at[pl.ds(s * R_A, R_A)],
                                out_hbm.at[c, which].at[pl.ds(s * R_A, R_A)])

            @pl.when(s == 15)
            def _ob():
                pltpu.sync_copy(acc_sh.at[pl.ds(15 * R_A, R_LAST)],
                                out_hbm.at[c, which].at[pl.ds(15 * R_A, R_LAST)])

        def scatter_ones(idxv):
            def fire(g, carry):
                pltpu.async_copy(onesv, acc_sh.at[idxv.at[g]], sem, add=True)
                return carry

            lax.fori_loop(0, NCHUNK, fire, 0)

            def drain(g, carry):
                pltpu.make_async_copy(onesv, acc_sh.at[idxv.at[0]],
                                      sem).wait()
                return carry

            lax.fori_loop(0, NCHUNK, drain, 0)

        zero_mine()
        plsc.subcore_barrier()
        scatter_ones(srcv)
        plsc.subcore_barrier()
        copy_mine(0)
        zero_mine()
        plsc.subcore_barrier()
        scatter_ones(dstv)
        plsc.subcore_barrier()
        copy_mine(1)

    return body(src3, dst3, ones_k, zeros_nf)


def _agg_partials(hw, src3, dst3, zeros_nf, F):
    @functools.partial(
        pl.kernel,
        out_type=jax.ShapeDtypeStruct((NC, N, F), jnp.float32),
        mesh=_MESH,
        scratch_types=[
            pltpu.VMEM((PH, K), jnp.int32),
            pltpu.VMEM((PH, K), jnp.int32),
            pltpu.VMEM((2, K, F), jnp.float32),
            pltpu.VMEM_SHARED((NP, F), jnp.float32),
            pltpu.SemaphoreType.DMA,
            pltpu.SemaphoreType.DMA,
        ],
    )
    def body(hw_hbm, src_hbm, dst_hbm, zeros_hbm, out_hbm,
             srcv, dstv, rows, acc_sh, gsem, ssem):
        c = lax.axis_index("c")
        s = lax.axis_index("s")
        wid = c * NS + s

        @pl.when(s < 15)
        def _zero_a():
            pltpu.sync_copy(zeros_hbm.at[pl.ds(s * R_A, R_A)],
                            acc_sh.at[pl.ds(s * R_A, R_A)])

        @pl.when(s == 15)
        def _zero_b():
            pltpu.sync_copy(zeros_hbm.at[pl.ds(15 * R_A, R_LAST)],
                            acc_sh.at[pl.ds(15 * R_A, R_LAST)])

        plsc.subcore_barrier()

        for p in range(NCHUNK // PH):
            pltpu.sync_copy(src_hbm.at[wid].at[pl.ds(p * PH, PH)], srcv)
            pltpu.sync_copy(dst_hbm.at[wid].at[pl.ds(p * PH, PH)], dstv)
            pltpu.async_copy(hw_hbm.at[srcv.at[0]], rows.at[0], gsem)

            def step(g, carry):
                b = lax.rem(g, 2)
                pltpu.make_async_copy(hw_hbm.at[srcv.at[0]],
                                      rows.at[b], gsem).wait()

                @pl.when(g >= 1)
                def _drain_prev():
                    pltpu.make_async_copy(rows.at[0],
                                          acc_sh.at[dstv.at[0]], ssem).wait()

                @pl.when(g + 1 < PH)
                def _next():
                    pltpu.async_copy(hw_hbm.at[srcv.at[g + 1]],
                                     rows.at[1 - b], gsem)

                pltpu.async_copy(rows.at[b], acc_sh.at[dstv.at[g]],
                                 ssem, add=True)
                return carry

            lax.fori_loop(0, PH, step, 0)
            pltpu.make_async_copy(rows.at[0], acc_sh.at[dstv.at[0]],
                                  ssem).wait()
        plsc.subcore_barrier()

        @pl.when(s < 15)
        def _out_a():
            pltpu.sync_copy(acc_sh.at[pl.ds(s * R_A, R_A)],
                            out_hbm.at[c].at[pl.ds(s * R_A, R_A)])

        @pl.when(s == 15)
        def _out_b():
            pltpu.sync_copy(acc_sh.at[pl.ds(15 * R_A, R_LAST)],
                            out_hbm.at[c].at[pl.ds(15 * R_A, R_LAST)])

    return body(hw, src3, dst3, zeros_nf)


def _dense_first(degT, featp, W1):
    def body(degT_ref, feat_ref, w1_ref, ns_ref, nd_ref, hw_ref):
        deg_out = degT_ref[0, :, 0:1] + degT_ref[2, :, 0:1]
        deg_in = degT_ref[1, :, 0:1] + degT_ref[3, :, 0:1]
        ns = lax.rsqrt(jnp.maximum(deg_out, 1.0))
        nd = lax.rsqrt(jnp.maximum(deg_in, 1.0))
        ns_b = jnp.broadcast_to(ns, (N, H))
        nd_b = jnp.broadcast_to(nd, (N, H))
        ns_ref[...] = ns_b
        nd_ref[...] = nd_b
        feat = feat_ref[...][:N]
        hw_ref[...] = jnp.concatenate(
            [jnp.dot(feat * ns_b, w1_ref[...],
                     preferred_element_type=jnp.float32),
             jnp.zeros((PAD_R, H), jnp.float32)], axis=0)

    return pl.pallas_call(
        body,
        out_shape=(
            jax.ShapeDtypeStruct((N, H), jnp.float32),
            jax.ShapeDtypeStruct((N, H), jnp.float32),
            jax.ShapeDtypeStruct((NP, H), jnp.float32),
        ),
    )(degT, featp, W1)


def _dense_mid(aggp, ns_b, nd_b, W):
    Fo = W.shape[1]

    def body(aggp_ref, ns_ref, nd_ref, w_ref, out_ref):
        agg = aggp_ref[0] + aggp_ref[1]
        h = jax.nn.relu(agg * nd_ref[...]) * ns_ref[...]
        out_ref[...] = jnp.concatenate(
            [jnp.dot(h, w_ref[...], preferred_element_type=jnp.float32),
             jnp.zeros((PAD_R, Fo), jnp.float32)], axis=0)

    return pl.pallas_call(
        body,
        out_shape=jax.ShapeDtypeStruct((NP, Fo), jnp.float32),
    )(aggp, ns_b, nd_b, W)


def _dense_act(aggp, ns_b, nd_b):
    def body(aggp_ref, ns_ref, nd_ref, out_ref):
        agg = aggp_ref[0] + aggp_ref[1]
        h = jax.nn.relu(agg * nd_ref[...]) * ns_ref[...]
        out_ref[...] = jnp.concatenate(
            [h, jnp.zeros((PAD_R, H), jnp.float32)], axis=0)

    return pl.pallas_call(
        body,
        out_shape=jax.ShapeDtypeStruct((NP, H), jnp.float32),
    )(aggp, ns_b, nd_b)


def _dense_out(aggp, nd_b, W3, b3):
    def body(aggp_ref, nd_ref, w_ref, b_ref, out_ref):
        agg = aggp_ref[0] + aggp_ref[1]
        mm = jnp.dot(agg, w_ref[...], preferred_element_type=jnp.float32)
        out_ref[...] = mm * nd_ref[:, :C] + b_ref[...][None, :]

    return pl.pallas_call(
        body,
        out_shape=jax.ShapeDtypeStruct((N, C), jnp.float32),
    )(aggp, nd_b, W3, b3)


def kernel(feat, edge_index, W1, W2, W3, b3):
    pad = N + (jnp.arange(EP - E, dtype=jnp.int32) % PAD_R)
    src3 = jnp.concatenate([edge_index[0], pad]).reshape(NW, NCHUNK, K)
    dst3 = jnp.concatenate([edge_index[1], pad]).reshape(NW, NCHUNK, K)
    featp = jnp.concatenate([feat, jnp.zeros((PAD_R, D), jnp.float32)])

    ones_k = jnp.ones((K, H), jnp.float32)
    z128 = jnp.zeros((N, H), jnp.float32)

    degp = _deg_partials(src3, dst3, ones_k, z128)
    degT = degp.reshape(2 * 2, N, H)
    ns_b, nd_b, hw1 = _dense_first(degT, featp, W1)
    aggp1 = _agg_partials(hw1, src3, dst3, z128, H)
    hw2 = _dense_mid(aggp1, ns_b, nd_b, W2)
    aggp2 = _agg_partials(hw2, src3, dst3, z128, H)
    u3 = _dense_act(aggp2, ns_b, nd_b)
    aggp3 = _agg_partials(u3, src3, dst3, z128, H)
    return _dense_out(aggp3, nd_b, W3, b3)
